# Initial kernel scaffold; baseline (speedup 1.0000x reference)
import jax
import jax.numpy as jnp
from jax.experimental import pallas as pl


def kernel(x, edge_index, edge_weight, W0a, b0a, W0b, b0b, W1a, b1a, W1b, b1b, W2a, b2a, W2b, b2b, Wl1, bl1, Wl2, bl2, eps0, gamma0, beta0, eps1, gamma1, beta1, eps2, gamma2, beta2):
    raise NotImplementedError("write your pallas kernel here")


# speedup vs baseline: 11.6960x; 11.6960x over previous
import functools

import jax
import jax.numpy as jnp
from jax import lax
from jax.experimental import pallas as pl
from jax.experimental.pallas import tpu as pltpu
from jax.experimental.pallas import tpu_sc as plsc

N_NODES = 10000
HID = 16
NC, NS = 2, 16
NW = NC * NS
CH = 128
NPAD = 10112
RPT = NPAD // NS


def _make_segsum(kc):
    mesh = plsc.VectorSubcoreMesh(core_axis_name="c", subcore_axis_name="s")

    @functools.partial(
        pl.kernel,
        mesh=mesh,
        compiler_params=pltpu.CompilerParams(use_tc_tiling_on_sc=False),
        out_type=jax.ShapeDtypeStruct((NC, NPAD, HID), jnp.float32),
        scratch_types=[
            pltpu.VMEM((kc, CH), jnp.int32),
            pltpu.VMEM((kc, CH), jnp.int32),
            pltpu.VMEM((CH, HID), jnp.float32),
            pltpu.VMEM_SHARED((NPAD, HID), jnp.float32),
            pltpu.SemaphoreType.DMA,
        ],
    )
    def segsum(z_hbm, zero_hbm, src_hbm, dst_hbm, out_hbm,
               src_v, dst_v, rows_v, acc_sh, sem):
        c = lax.axis_index("c")
        s = lax.axis_index("s")
        wid = c * NS + s
        nbase = s * RPT
        pltpu.sync_copy(zero_hbm.at[pl.ds(nbase, RPT)],
                        acc_sh.at[pl.ds(nbase, RPT)])
        pltpu.sync_copy(src_hbm.at[wid], src_v)
        pltpu.sync_copy(dst_hbm.at[wid], dst_v)
        plsc.subcore_barrier()

        def chunk(j, carry):
            pltpu.async_copy(z_hbm.at[src_v.at[j]], rows_v, sem).wait()
            pltpu.sync_copy(rows_v, acc_sh.at[dst_v.at[j]], add=True)
            return carry

        lax.fori_loop(0, kc, chunk, 0)
        plsc.subcore_barrier()
        pltpu.sync_copy(acc_sh.at[pl.ds(nbase, RPT)],
                        out_hbm.at[c].at[pl.ds(nbase, RPT)])

    return segsum


def _mm_body(x_ref, w_ref, o_ref):
    o_ref[...] = jnp.dot(x_ref[...], w_ref[...],
                         preferred_element_type=jnp.float32)


def _bn(v, g, b):
    m = jnp.mean(v, axis=0, keepdims=True)
    var = jnp.mean((v - m) ** 2, axis=0, keepdims=True)
    return (v - m) * lax.rsqrt(var + 1e-5) * g + b


def _layer0_body(y_ref, agg_ref, eps_ref, ba_ref, wb_ref, bb_ref, g_ref,
                 be_ref, o_ref):
    agg = agg_ref[0, :N_NODES, :] + agg_ref[1, :N_NODES, :]
    t = jnp.maximum((1.0 + eps_ref[0, 0]) * y_ref[...] + agg + ba_ref[...],
                    0.0)
    v = jnp.maximum(
        jnp.dot(t, wb_ref[...], preferred_element_type=jnp.float32)
        + bb_ref[...], 0.0)
    o_ref[...] = _bn(v, g_ref[...], be_ref[...])


def _layer_body(h_ref, agg_ref, eps_ref, wa_ref, ba_ref, wb_ref, bb_ref,
                g_ref, be_ref, o_ref):
    agg = agg_ref[0, :N_NODES, :] + agg_ref[1, :N_NODES, :]
    m = (1.0 + eps_ref[0, 0]) * h_ref[...] + agg
    t = jnp.maximum(
        jnp.dot(m, wa_ref[...], preferred_element_type=jnp.float32)
        + ba_ref[...], 0.0)
    v = jnp.maximum(
        jnp.dot(t, wb_ref[...], preferred_element_type=jnp.float32)
        + bb_ref[...], 0.0)
    o_ref[...] = _bn(v, g_ref[...], be_ref[...])


def _layer2_head_body(h_ref, agg_ref, eps_ref, wa_ref, ba_ref, wb_ref,
                      bb_ref, g_ref, be_ref, wl1_ref, bl1_ref, wl2_ref,
                      bl2_ref, o_ref):
    agg = agg_ref[0, :N_NODES, :] + agg_ref[1, :N_NODES, :]
    m = (1.0 + eps_ref[0, 0]) * h_ref[...] + agg
    t = jnp.maximum(
        jnp.dot(m, wa_ref[...], preferred_element_type=jnp.float32)
        + ba_ref[...], 0.0)
    v = jnp.maximum(
        jnp.dot(t, wb_ref[...], preferred_element_type=jnp.float32)
        + bb_ref[...], 0.0)
    h = _bn(v, g_ref[...], be_ref[...])
    t2 = jnp.maximum(
        jnp.dot(h, wl1_ref[...], preferred_element_type=jnp.float32)
        + bl1_ref[...], 0.0)
    sc = jnp.dot(t2, wl2_ref[...], preferred_element_type=jnp.float32) \
        + bl2_ref[...]
    mx = jnp.max(sc, axis=-1, keepdims=True)
    e = jnp.exp(sc - mx)
    o_ref[...] = sc - mx - jnp.log(jnp.sum(e, axis=-1, keepdims=True))


def kernel(x, edge_index, edge_weight, W0a, b0a, W0b, b0b, W1a, b1a, W1b,
           b1b, W2a, b2a, W2b, b2b, Wl1, bl1, Wl2, bl2, eps0, gamma0, beta0,
           eps1, gamma1, beta1, eps2, gamma2, beta2):
    n, f = x.shape
    e = edge_index.shape[1]
    c = Wl2.shape[1]

    kc = -(-e // (NW * CH))
    epad = NW * kc * CH - e
    src = edge_index[0].astype(jnp.int32)
    dst = edge_index[1].astype(jnp.int32)
    src_p = jnp.concatenate([src, jnp.zeros((epad,), jnp.int32)])
    dst_p = jnp.concatenate([dst, jnp.full((epad,), n, jnp.int32)])
    src_p = src_p.reshape(NW, kc, CH)
    dst_p = dst_p.reshape(NW, kc, CH)
    zeros_pad = jnp.zeros((NPAD, HID), jnp.float32)

    r2 = lambda a: a.reshape(1, -1)

    mm = pl.pallas_call(
        _mm_body, out_shape=jax.ShapeDtypeStruct((n, HID), jnp.float32))
    layer0 = pl.pallas_call(
        _layer0_body, out_shape=jax.ShapeDtypeStruct((n, HID), jnp.float32))
    layer = pl.pallas_call(
        _layer_body, out_shape=jax.ShapeDtypeStruct((n, HID), jnp.float32))
    layer2_head = pl.pallas_call(
        _layer2_head_body, out_shape=jax.ShapeDtypeStruct((n, c),
                                                          jnp.float32))
    segsum = _make_segsum(kc)

    y = mm(x, W0a)
    agg0 = segsum(y, zeros_pad, src_p, dst_p)
    h0 = layer0(y, agg0, jnp.reshape(eps0, (1, 1)), r2(b0a), W0b, r2(b0b),
                r2(gamma0), r2(beta0))
    agg1 = segsum(h0, zeros_pad, src_p, dst_p)
    h1 = layer(h0, agg1, jnp.reshape(eps1, (1, 1)), W1a, r2(b1a), W1b,
               r2(b1b), r2(gamma1), r2(beta1))
    agg2 = segsum(h1, zeros_pad, src_p, dst_p)
    out = layer2_head(h1, agg2, jnp.reshape(eps2, (1, 1)), W2a, r2(b2a),
                      W2b, r2(b2b), r2(gamma2), r2(beta2), Wl1, r2(bl1),
                      Wl2, r2(bl2))
    return out


# speedup vs baseline: 14.2336x; 1.2170x over previous
import functools

import jax
import jax.numpy as jnp
from jax import lax
from jax.experimental import pallas as pl
from jax.experimental.pallas import tpu as pltpu
from jax.experimental.pallas import tpu_sc as plsc

N_NODES = 10000
HID = 16
NC, NS = 2, 16
NW = NC * NS
CH = 128
NPAD = 10112
RPT = NPAD // NS


def _make_segsum(kc):
    mesh = plsc.VectorSubcoreMesh(core_axis_name="c", subcore_axis_name="s")

    @functools.partial(
        pl.kernel,
        mesh=mesh,
        compiler_params=pltpu.CompilerParams(use_tc_tiling_on_sc=False),
        out_type=jax.ShapeDtypeStruct((NC, NPAD, HID), jnp.float32),
        scratch_types=[
            pltpu.VMEM((kc, CH), jnp.int32),
            pltpu.VMEM((kc, CH), jnp.int32),
            pltpu.VMEM((CH, HID), jnp.float32),
            pltpu.VMEM((CH, HID), jnp.float32),
            pltpu.VMEM_SHARED((NPAD, HID), jnp.float32),
            pltpu.SemaphoreType.DMA,
            pltpu.SemaphoreType.DMA,
        ],
    )
    def segsum(z_hbm, zero_hbm, src_hbm, dst_hbm, out_hbm,
               src_v, dst_v, rows0_v, rows1_v, acc_sh, sem0, sem1):
        c = lax.axis_index("c")
        s = lax.axis_index("s")
        wid = c * NS + s
        nbase = s * RPT
        pltpu.sync_copy(zero_hbm.at[pl.ds(nbase, RPT)],
                        acc_sh.at[pl.ds(nbase, RPT)])
        pltpu.sync_copy(src_hbm.at[wid], src_v)
        pltpu.sync_copy(dst_hbm.at[wid], dst_v)
        plsc.subcore_barrier()

        pltpu.async_copy(z_hbm.at[src_v.at[0]], rows0_v, sem0)

        def chunk2(i, carry):
            g = 2 * i
            pltpu.async_copy(z_hbm.at[src_v.at[g + 1]], rows1_v, sem1)
            pltpu.make_async_copy(z_hbm.at[src_v.at[g]], rows0_v, sem0).wait()
            pltpu.sync_copy(rows0_v, acc_sh.at[dst_v.at[g]], add=True)
            pltpu.async_copy(z_hbm.at[src_v.at[g + 2]], rows0_v, sem0)
            pltpu.make_async_copy(
                z_hbm.at[src_v.at[g + 1]], rows1_v, sem1).wait()
            pltpu.sync_copy(rows1_v, acc_sh.at[dst_v.at[g + 1]], add=True)
            return carry

        lax.fori_loop(0, (kc - 2) // 2, chunk2, 0, unroll=False)
        pltpu.async_copy(z_hbm.at[src_v.at[kc - 1]], rows1_v, sem1)
        pltpu.make_async_copy(
            z_hbm.at[src_v.at[kc - 2]], rows0_v, sem0).wait()
        pltpu.sync_copy(rows0_v, acc_sh.at[dst_v.at[kc - 2]], add=True)
        pltpu.make_async_copy(
            z_hbm.at[src_v.at[kc - 1]], rows1_v, sem1).wait()
        pltpu.sync_copy(rows1_v, acc_sh.at[dst_v.at[kc - 1]], add=True)
        plsc.subcore_barrier()
        pltpu.sync_copy(acc_sh.at[pl.ds(nbase, RPT)],
                        out_hbm.at[c].at[pl.ds(nbase, RPT)])

    return segsum


def _mm_body(x_ref, w_ref, o_ref):
    o_ref[...] = jnp.dot(x_ref[...], w_ref[...],
                         preferred_element_type=jnp.float32)


def _bn(v, g, b):
    m = jnp.mean(v, axis=0, keepdims=True)
    var = jnp.mean((v - m) ** 2, axis=0, keepdims=True)
    return (v - m) * lax.rsqrt(var + 1e-5) * g + b


def _layer0_body(y_ref, agg_ref, eps_ref, ba_ref, wb_ref, bb_ref, g_ref,
                 be_ref, o_ref):
    agg = agg_ref[0, :N_NODES, :] + agg_ref[1, :N_NODES, :]
    t = jnp.maximum((1.0 + eps_ref[0, 0]) * y_ref[...] + agg + ba_ref[...],
                    0.0)
    v = jnp.maximum(
        jnp.dot(t, wb_ref[...], preferred_element_type=jnp.float32)
        + bb_ref[...], 0.0)
    o_ref[...] = _bn(v, g_ref[...], be_ref[...])


def _layer_body(h_ref, agg_ref, eps_ref, wa_ref, ba_ref, wb_ref, bb_ref,
                g_ref, be_ref, o_ref):
    agg = agg_ref[0, :N_NODES, :] + agg_ref[1, :N_NODES, :]
    m = (1.0 + eps_ref[0, 0]) * h_ref[...] + agg
    t = jnp.maximum(
        jnp.dot(m, wa_ref[...], preferred_element_type=jnp.float32)
        + ba_ref[...], 0.0)
    v = jnp.maximum(
        jnp.dot(t, wb_ref[...], preferred_element_type=jnp.float32)
        + bb_ref[...], 0.0)
    o_ref[...] = _bn(v, g_ref[...], be_ref[...])


def _layer2_head_body(h_ref, agg_ref, eps_ref, wa_ref, ba_ref, wb_ref,
                      bb_ref, g_ref, be_ref, wl1_ref, bl1_ref, wl2_ref,
                      bl2_ref, o_ref):
    agg = agg_ref[0, :N_NODES, :] + agg_ref[1, :N_NODES, :]
    m = (1.0 + eps_ref[0, 0]) * h_ref[...] + agg
    t = jnp.maximum(
        jnp.dot(m, wa_ref[...], preferred_element_type=jnp.float32)
        + ba_ref[...], 0.0)
    v = jnp.maximum(
        jnp.dot(t, wb_ref[...], preferred_element_type=jnp.float32)
        + bb_ref[...], 0.0)
    h = _bn(v, g_ref[...], be_ref[...])
    t2 = jnp.maximum(
        jnp.dot(h, wl1_ref[...], preferred_element_type=jnp.float32)
        + bl1_ref[...], 0.0)
    sc = jnp.dot(t2, wl2_ref[...], preferred_element_type=jnp.float32) \
        + bl2_ref[...]
    mx = jnp.max(sc, axis=-1, keepdims=True)
    e = jnp.exp(sc - mx)
    o_ref[...] = sc - mx - jnp.log(jnp.sum(e, axis=-1, keepdims=True))


def kernel(x, edge_index, edge_weight, W0a, b0a, W0b, b0b, W1a, b1a, W1b,
           b1b, W2a, b2a, W2b, b2b, Wl1, bl1, Wl2, bl2, eps0, gamma0, beta0,
           eps1, gamma1, beta1, eps2, gamma2, beta2):
    n, f = x.shape
    e = edge_index.shape[1]
    c = Wl2.shape[1]

    kc = -(-e // (NW * CH))
    kc = kc + (kc % 2)
    epad = NW * kc * CH - e
    src = edge_index[0].astype(jnp.int32)
    dst = edge_index[1].astype(jnp.int32)
    src_p = jnp.concatenate([src, jnp.zeros((epad,), jnp.int32)])
    dst_p = jnp.concatenate([dst, jnp.full((epad,), n, jnp.int32)])
    src_p = src_p.reshape(NW, kc, CH)
    dst_p = dst_p.reshape(NW, kc, CH)
    zeros_pad = jnp.zeros((NPAD, HID), jnp.float32)

    r2 = lambda a: a.reshape(1, -1)

    mm = pl.pallas_call(
        _mm_body, out_shape=jax.ShapeDtypeStruct((n, HID), jnp.float32))
    layer0 = pl.pallas_call(
        _layer0_body, out_shape=jax.ShapeDtypeStruct((n, HID), jnp.float32))
    layer = pl.pallas_call(
        _layer_body, out_shape=jax.ShapeDtypeStruct((n, HID), jnp.float32))
    layer2_head = pl.pallas_call(
        _layer2_head_body, out_shape=jax.ShapeDtypeStruct((n, c),
                                                          jnp.float32))
    segsum = _make_segsum(kc)

    y = mm(x, W0a)
    agg0 = segsum(y, zeros_pad, src_p, dst_p)
    h0 = layer0(y, agg0, jnp.reshape(eps0, (1, 1)), r2(b0a), W0b, r2(b0b),
                r2(gamma0), r2(beta0))
    agg1 = segsum(h0, zeros_pad, src_p, dst_p)
    h1 = layer(h0, agg1, jnp.reshape(eps1, (1, 1)), W1a, r2(b1a), W1b,
               r2(b1b), r2(gamma1), r2(beta1))
    agg2 = segsum(h1, zeros_pad, src_p, dst_p)
    out = layer2_head(h1, agg2, jnp.reshape(eps2, (1, 1)), W2a, r2(b2a),
                      W2b, r2(b2b), r2(gamma2), r2(beta2), Wl1, r2(bl1),
                      Wl2, r2(bl2))
    return out


# speedup vs baseline: 14.7205x; 1.0342x over previous
import functools

import jax
import jax.numpy as jnp
from jax import lax
from jax.experimental import pallas as pl
from jax.experimental.pallas import tpu as pltpu
from jax.experimental.pallas import tpu_sc as plsc

N_NODES = 10000
HID = 16
NC, NS = 2, 16
NW = NC * NS
CH = 128
NPAD = 10112
RPT = NPAD // NS
NB = 4
PF = 2


def _make_segsum(kc):
    mesh = plsc.VectorSubcoreMesh(core_axis_name="c", subcore_axis_name="s")

    @functools.partial(
        pl.kernel,
        mesh=mesh,
        compiler_params=pltpu.CompilerParams(use_tc_tiling_on_sc=False),
        out_type=jax.ShapeDtypeStruct((NC, NPAD, HID), jnp.float32),
        scratch_types=[
            pltpu.VMEM((kc, CH), jnp.int32),
            pltpu.VMEM((kc, CH), jnp.int32),
            [pltpu.VMEM((CH, HID), jnp.float32) for _ in range(NB)],
            pltpu.VMEM_SHARED((NPAD, HID), jnp.float32),
            [pltpu.SemaphoreType.DMA for _ in range(NB)],
            [pltpu.SemaphoreType.DMA for _ in range(NB)],
        ],
    )
    def segsum(z_hbm, zero_hbm, src_hbm, dst_hbm, out_hbm,
               src_v, dst_v, rows, acc_sh, gsem, ssem):
        c = lax.axis_index("c")
        s = lax.axis_index("s")
        wid = c * NS + s
        nbase = s * RPT
        pltpu.sync_copy(zero_hbm.at[pl.ds(nbase, RPT)],
                        acc_sh.at[pl.ds(nbase, RPT)])
        pltpu.sync_copy(src_hbm.at[wid], src_v)
        pltpu.sync_copy(dst_hbm.at[wid], dst_v)
        plsc.subcore_barrier()

        for g in range(PF):
            pltpu.async_copy(z_hbm.at[src_v.at[g]], rows[g % NB],
                             gsem[g % NB])

        def body(i, carry):
            for b in range(NB):
                g = i * NB + b
                bp = (b + PF) % NB

                @pl.when(g + PF >= NB)
                def _():
                    gp = g + PF - NB
                    pltpu.make_async_copy(
                        rows[bp], acc_sh.at[dst_v.at[gp]], ssem[bp]).wait()

                @pl.when(g + PF < kc)
                def _():
                    pltpu.async_copy(z_hbm.at[src_v.at[g + PF]], rows[bp],
                                     gsem[bp])

                pltpu.make_async_copy(z_hbm.at[src_v.at[g]], rows[b],
                                      gsem[b]).wait()
                pltpu.async_copy(rows[b], acc_sh.at[dst_v.at[g]], ssem[b],
                                 add=True)
            return carry

        lax.fori_loop(0, kc // NB, body, 0, unroll=False)
        for gg in range(kc - PF, kc):
            b = gg % NB
            pltpu.make_async_copy(rows[b], acc_sh.at[dst_v.at[gg]],
                                  ssem[b]).wait()
        plsc.subcore_barrier()
        pltpu.sync_copy(acc_sh.at[pl.ds(nbase, RPT)],
                        out_hbm.at[c].at[pl.ds(nbase, RPT)])

    return segsum


def _mm_body(x_ref, w_ref, o_ref):
    o_ref[...] = jnp.dot(x_ref[...], w_ref[...],
                         preferred_element_type=jnp.float32)


def _bn(v, g, b):
    m = jnp.mean(v, axis=0, keepdims=True)
    var = jnp.mean((v - m) ** 2, axis=0, keepdims=True)
    return (v - m) * lax.rsqrt(var + 1e-5) * g + b


def _layer0_body(y_ref, agg_ref, eps_ref, ba_ref, wb_ref, bb_ref, g_ref,
                 be_ref, o_ref):
    agg = agg_ref[0, :N_NODES, :] + agg_ref[1, :N_NODES, :]
    t = jnp.maximum((1.0 + eps_ref[0, 0]) * y_ref[...] + agg + ba_ref[...],
                    0.0)
    v = jnp.maximum(
        jnp.dot(t, wb_ref[...], preferred_element_type=jnp.float32)
        + bb_ref[...], 0.0)
    o_ref[...] = _bn(v, g_ref[...], be_ref[...])


def _layer_body(h_ref, agg_ref, eps_ref, wa_ref, ba_ref, wb_ref, bb_ref,
                g_ref, be_ref, o_ref):
    agg = agg_ref[0, :N_NODES, :] + agg_ref[1, :N_NODES, :]
    m = (1.0 + eps_ref[0, 0]) * h_ref[...] + agg
    t = jnp.maximum(
        jnp.dot(m, wa_ref[...], preferred_element_type=jnp.float32)
        + ba_ref[...], 0.0)
    v = jnp.maximum(
        jnp.dot(t, wb_ref[...], preferred_element_type=jnp.float32)
        + bb_ref[...], 0.0)
    o_ref[...] = _bn(v, g_ref[...], be_ref[...])


def _layer2_head_body(h_ref, agg_ref, eps_ref, wa_ref, ba_ref, wb_ref,
                      bb_ref, g_ref, be_ref, wl1_ref, bl1_ref, wl2_ref,
                      bl2_ref, o_ref):
    agg = agg_ref[0, :N_NODES, :] + agg_ref[1, :N_NODES, :]
    m = (1.0 + eps_ref[0, 0]) * h_ref[...] + agg
    t = jnp.maximum(
        jnp.dot(m, wa_ref[...], preferred_element_type=jnp.float32)
        + ba_ref[...], 0.0)
    v = jnp.maximum(
        jnp.dot(t, wb_ref[...], preferred_element_type=jnp.float32)
        + bb_ref[...], 0.0)
    h = _bn(v, g_ref[...], be_ref[...])
    t2 = jnp.maximum(
        jnp.dot(h, wl1_ref[...], preferred_element_type=jnp.float32)
        + bl1_ref[...], 0.0)
    sc = jnp.dot(t2, wl2_ref[...], preferred_element_type=jnp.float32) \
        + bl2_ref[...]
    mx = jnp.max(sc, axis=-1, keepdims=True)
    e = jnp.exp(sc - mx)
    o_ref[...] = sc - mx - jnp.log(jnp.sum(e, axis=-1, keepdims=True))


def kernel(x, edge_index, edge_weight, W0a, b0a, W0b, b0b, W1a, b1a, W1b,
           b1b, W2a, b2a, W2b, b2b, Wl1, bl1, Wl2, bl2, eps0, gamma0, beta0,
           eps1, gamma1, beta1, eps2, gamma2, beta2):
    n, f = x.shape
    e = edge_index.shape[1]
    c = Wl2.shape[1]

    kc = -(-e // (NW * CH))
    kc = -(-kc // NB) * NB
    epad = NW * kc * CH - e
    src = edge_index[0].astype(jnp.int32)
    dst = edge_index[1].astype(jnp.int32)
    src_p = jnp.concatenate([src, jnp.zeros((epad,), jnp.int32)])
    dst_p = jnp.concatenate([dst, jnp.full((epad,), n, jnp.int32)])
    src_p = src_p.reshape(NW, kc, CH)
    dst_p = dst_p.reshape(NW, kc, CH)
    zeros_pad = jnp.zeros((NPAD, HID), jnp.float32)

    r2 = lambda a: a.reshape(1, -1)

    mm = pl.pallas_call(
        _mm_body, out_shape=jax.ShapeDtypeStruct((n, HID), jnp.float32))
    layer0 = pl.pallas_call(
        _layer0_body, out_shape=jax.ShapeDtypeStruct((n, HID), jnp.float32))
    layer = pl.pallas_call(
        _layer_body, out_shape=jax.ShapeDtypeStruct((n, HID), jnp.float32))
    layer2_head = pl.pallas_call(
        _layer2_head_body, out_shape=jax.ShapeDtypeStruct((n, c),
                                                          jnp.float32))
    segsum = _make_segsum(kc)

    y = mm(x, W0a)
    agg0 = segsum(y, zeros_pad, src_p, dst_p)
    h0 = layer0(y, agg0, jnp.reshape(eps0, (1, 1)), r2(b0a), W0b, r2(b0b),
                r2(gamma0), r2(beta0))
    agg1 = segsum(h0, zeros_pad, src_p, dst_p)
    h1 = layer(h0, agg1, jnp.reshape(eps1, (1, 1)), W1a, r2(b1a), W1b,
               r2(b1b), r2(gamma1), r2(beta1))
    agg2 = segsum(h1, zeros_pad, src_p, dst_p)
    out = layer2_head(h1, agg2, jnp.reshape(eps2, (1, 1)), W2a, r2(b2a),
                      W2b, r2(b2b), r2(gamma2), r2(beta2), Wl1, r2(bl1),
                      Wl2, r2(bl2))
    return out


# speedup vs baseline: 20.7538x; 1.4099x over previous
import functools

import jax
import jax.numpy as jnp
from jax import lax
from jax.experimental import pallas as pl
from jax.experimental.pallas import tpu as pltpu
from jax.experimental.pallas import tpu_sc as plsc

N_NODES = 10000
HID = 16
NC, NS = 2, 16
NW = NC * NS
CH = 128
NPAD = 10112
RPT = NPAD // NS
NB = 4
PF = 2


def _make_segsum(kc, ch):
    mesh = plsc.VectorSubcoreMesh(core_axis_name="c", subcore_axis_name="s")

    @functools.partial(
        pl.kernel,
        mesh=mesh,
        compiler_params=pltpu.CompilerParams(use_tc_tiling_on_sc=False),
        out_type=jax.ShapeDtypeStruct((NC, NPAD, HID), jnp.float32),
        scratch_types=[
            pltpu.VMEM((kc, ch), jnp.int32),
            pltpu.VMEM((kc, ch), jnp.int32),
            [pltpu.VMEM((ch, HID), jnp.float32) for _ in range(NB)],
            pltpu.VMEM_SHARED((NPAD, HID), jnp.float32),
            [pltpu.SemaphoreType.DMA for _ in range(NB)],
            [pltpu.SemaphoreType.DMA for _ in range(NB)],
        ],
    )
    def segsum(z_hbm, zero_hbm, src_hbm, dst_hbm, out_hbm,
               src_v, dst_v, rows, acc_sh, gsem, ssem):
        c = lax.axis_index("c")
        s = lax.axis_index("s")
        wid = c * NS + s
        nbase = s * RPT
        pltpu.sync_copy(zero_hbm.at[pl.ds(nbase, RPT)],
                        acc_sh.at[pl.ds(nbase, RPT)])
        pltpu.sync_copy(src_hbm.at[wid], src_v)
        pltpu.sync_copy(dst_hbm.at[wid], dst_v)
        plsc.subcore_barrier()

        for g in range(PF):
            pltpu.async_copy(z_hbm.at[src_v.at[g]], rows[g % NB],
                             gsem[g % NB])

        def body(i, carry):
            for b in range(NB):
                g = i * NB + b
                bp = (b + PF) % NB

                @pl.when(g + PF >= NB)
                def _():
                    gp = g + PF - NB
                    pltpu.make_async_copy(
                        rows[bp], acc_sh.at[dst_v.at[gp]], ssem[bp]).wait()

                @pl.when(g + PF < kc)
                def _():
                    pltpu.async_copy(z_hbm.at[src_v.at[g + PF]], rows[bp],
                                     gsem[bp])

                pltpu.make_async_copy(z_hbm.at[src_v.at[g]], rows[b],
                                      gsem[b]).wait()
                pltpu.async_copy(rows[b], acc_sh.at[dst_v.at[g]], ssem[b],
                                 add=True)
            return carry

        lax.fori_loop(0, kc // NB, body, 0, unroll=False)
        for gg in range(kc - PF, kc):
            b = gg % NB
            pltpu.make_async_copy(rows[b], acc_sh.at[dst_v.at[gg]],
                                  ssem[b]).wait()
        plsc.subcore_barrier()
        pltpu.sync_copy(acc_sh.at[pl.ds(nbase, RPT)],
                        out_hbm.at[c].at[pl.ds(nbase, RPT)])

    return segsum


def _mm_body(x_ref, w_ref, o_ref):
    o_ref[...] = jnp.dot(x_ref[...], w_ref[...],
                         preferred_element_type=jnp.float32)


def _bn(v, g, b):
    m = jnp.mean(v, axis=0, keepdims=True)
    var = jnp.mean((v - m) ** 2, axis=0, keepdims=True)
    return (v - m) * lax.rsqrt(var + 1e-5) * g + b


def _layer0_body(y_ref, agg_ref, eps_ref, ba_ref, wb_ref, bb_ref, g_ref,
                 be_ref, o_ref):
    agg = agg_ref[0, :N_NODES, :] + agg_ref[1, :N_NODES, :]
    t = jnp.maximum((1.0 + eps_ref[0, 0]) * y_ref[...] + agg + ba_ref[...],
                    0.0)
    v = jnp.maximum(
        jnp.dot(t, wb_ref[...], preferred_element_type=jnp.float32)
        + bb_ref[...], 0.0)
    o_ref[...] = _bn(v, g_ref[...], be_ref[...])


def _layer_body(h_ref, agg_ref, eps_ref, wa_ref, ba_ref, wb_ref, bb_ref,
                g_ref, be_ref, o_ref):
    agg = agg_ref[0, :N_NODES, :] + agg_ref[1, :N_NODES, :]
    m = (1.0 + eps_ref[0, 0]) * h_ref[...] + agg
    t = jnp.maximum(
        jnp.dot(m, wa_ref[...], preferred_element_type=jnp.float32)
        + ba_ref[...], 0.0)
    v = jnp.maximum(
        jnp.dot(t, wb_ref[...], preferred_element_type=jnp.float32)
        + bb_ref[...], 0.0)
    o_ref[...] = _bn(v, g_ref[...], be_ref[...])


def _layer2_head_body(h_ref, agg_ref, eps_ref, wa_ref, ba_ref, wb_ref,
                      bb_ref, g_ref, be_ref, wl1_ref, bl1_ref, wl2_ref,
                      bl2_ref, o_ref):
    agg = agg_ref[0, :N_NODES, :] + agg_ref[1, :N_NODES, :]
    m = (1.0 + eps_ref[0, 0]) * h_ref[...] + agg
    t = jnp.maximum(
        jnp.dot(m, wa_ref[...], preferred_element_type=jnp.float32)
        + ba_ref[...], 0.0)
    v = jnp.maximum(
        jnp.dot(t, wb_ref[...], preferred_element_type=jnp.float32)
        + bb_ref[...], 0.0)
    h = _bn(v, g_ref[...], be_ref[...])
    t2 = jnp.maximum(
        jnp.dot(h, wl1_ref[...], preferred_element_type=jnp.float32)
        + bl1_ref[...], 0.0)
    sc = jnp.dot(t2, wl2_ref[...], preferred_element_type=jnp.float32) \
        + bl2_ref[...]
    mx = jnp.max(sc, axis=-1, keepdims=True)
    e = jnp.exp(sc - mx)
    o_ref[...] = sc - mx - jnp.log(jnp.sum(e, axis=-1, keepdims=True))


def kernel(x, edge_index, edge_weight, W0a, b0a, W0b, b0b, W1a, b1a, W1b,
           b1b, W2a, b2a, W2b, b2b, Wl1, bl1, Wl2, bl2, eps0, gamma0, beta0,
           eps1, gamma1, beta1, eps2, gamma2, beta2):
    n, f = x.shape
    e = edge_index.shape[1]
    c = Wl2.shape[1]

    ch = next((q for q in (128, 125, 120, 100, 80, 64, 50, 40, 32, 16)
               if e % (NW * q * NB) == 0), None)
    src = edge_index[0].astype(jnp.int32)
    dst = edge_index[1].astype(jnp.int32)
    if ch is not None:
        kc = e // (NW * ch)
        src_p = src.reshape(NW, kc, ch)
        dst_p = dst.reshape(NW, kc, ch)
    else:
        ch = CH
        kc = -(-e // (NW * ch))
        kc = -(-kc // NB) * NB
        epad = NW * kc * ch - e
        src_p = jnp.concatenate(
            [src, jnp.zeros((epad,), jnp.int32)]).reshape(NW, kc, ch)
        dst_p = jnp.concatenate(
            [dst, jnp.full((epad,), n, jnp.int32)]).reshape(NW, kc, ch)
    zeros_pad = jnp.zeros((NPAD, HID), jnp.float32)

    r2 = lambda a: a.reshape(1, -1)

    mm = pl.pallas_call(
        _mm_body, out_shape=jax.ShapeDtypeStruct((n, HID), jnp.float32))
    layer0 = pl.pallas_call(
        _layer0_body, out_shape=jax.ShapeDtypeStruct((n, HID), jnp.float32))
    layer = pl.pallas_call(
        _layer_body, out_shape=jax.ShapeDtypeStruct((n, HID), jnp.float32))
    layer2_head = pl.pallas_call(
        _layer2_head_body, out_shape=jax.ShapeDtypeStruct((n, c),
                                                          jnp.float32))
    segsum = _make_segsum(kc, ch)

    y = mm(x, W0a)
    agg0 = segsum(y, zeros_pad, src_p, dst_p)
    h0 = layer0(y, agg0, jnp.reshape(eps0, (1, 1)), r2(b0a), W0b, r2(b0b),
                r2(gamma0), r2(beta0))
    agg1 = segsum(h0, zeros_pad, src_p, dst_p)
    h1 = layer(h0, agg1, jnp.reshape(eps1, (1, 1)), W1a, r2(b1a), W1b,
               r2(b1b), r2(gamma1), r2(beta1))
    agg2 = segsum(h1, zeros_pad, src_p, dst_p)
    out = layer2_head(h1, agg2, jnp.reshape(eps2, (1, 1)), W2a, r2(b2a),
                      W2b, r2(b2b), r2(gamma2), r2(beta2), Wl1, r2(bl1),
                      Wl2, r2(bl2))
    return out
